# baseline (speedup 1.0000x reference)
import functools
import math

import jax
import jax.numpy as jnp
from jax import lax
from jax.experimental import pallas as pl
from jax.experimental.pallas import tpu as pltpu
from jax.experimental.pallas import tpu_sc as plsc

_HID = 64
_SCALE = math.sqrt(_HID)
_NC = 2
_NS = 16
_NW = _NC * _NS
_LANES = 16
_CHUNK = 128


def _sc_embed(steps: int):
    mesh = plsc.VectorSubcoreMesh(core_axis_name="c", subcore_axis_name="s")

    @functools.partial(
        pl.kernel,
        mesh=mesh,
        out_type=jax.ShapeDtypeStruct((_NW, steps, _CHUNK, _HID), jnp.float32),
        scratch_types=[
            pltpu.VMEM((steps, _CHUNK), jnp.int32),
            pltpu.VMEM((2, _CHUNK, _HID), jnp.float32),
            pltpu.SemaphoreType.DMA,
            pltpu.SemaphoreType.DMA,
        ],
        compiler_params=pltpu.CompilerParams(use_tc_tiling_on_sc=False),
    )
    def k(ids_hbm, table_hbm, out_hbm, idx_v, rows_v, sem0, sem1):
        wid = lax.axis_index("s") * _NC + lax.axis_index("c")
        pltpu.sync_copy(ids_hbm.at[wid], idx_v)

        sems = (sem0, sem1)

        def start(s, b):
            pltpu.async_copy(table_hbm.at[idx_v.at[s]], rows_v.at[b], sems[b])

        def finish(s, b):
            pltpu.make_async_copy(
                table_hbm.at[idx_v.at[s]], rows_v.at[b], sems[b]
            ).wait()

        def scale_and_store(s, b):
            def row(r, _):
                for j in range(_HID // _LANES):
                    sl = pl.ds(j * _LANES, _LANES)
                    rows_v[b, r, sl] = rows_v[b, r, sl] * _SCALE
                return _

            lax.fori_loop(0, _CHUNK, row, 0, unroll=4)
            pltpu.sync_copy(rows_v.at[b], out_hbm.at[wid, s])

        start(0, 0)

        def body(g, carry):
            for b in range(2):
                s = g + b
                finish(s, b)

                @pl.when(s + 1 < steps)
                def _prefetch(s=s, b=b):
                    start(s + 1, 1 - b)

                scale_and_store(s, b)
            return carry

        lax.fori_loop(0, steps // 2, lambda i, c: body(i * 2, c), 0)

    return k


def kernel(input_ids, embed_table):
    b, l = input_ids.shape
    total = b * l
    per_w = total // _NW
    steps = per_w // _CHUNK
    assert per_w * _NW == total and steps * _CHUNK == per_w
    ids = input_ids.reshape(_NW, steps, _CHUNK).astype(jnp.int32)
    out = _sc_embed(steps)(ids, embed_table)
    return out.reshape(b, l, _HID)
